# Initial kernel scaffold; baseline (speedup 1.0000x reference)
import jax
import jax.numpy as jnp
from jax.experimental import pallas as pl


def kernel(x, edge_index, edge_attr, W):
    raise NotImplementedError("write your pallas kernel here")


# speedup vs baseline: 19.4695x; 19.4695x over previous
import functools

import jax
import jax.numpy as jnp
from jax import lax
from jax.experimental import pallas as pl
from jax.experimental.pallas import tpu as pltpu
from jax.experimental.pallas import tpu_sc as plsc

_NC = 2
_NS = 16
_LANES = 16


def _make_deg_kernel(E, NPAD):
    EPT = E // _NS
    CH = 80
    ITERS = EPT // CH
    SPT = NPAD // _NS
    mesh = plsc.VectorSubcoreMesh(core_axis_name="c", subcore_axis_name="s")

    @functools.partial(
        pl.kernel,
        out_type=jax.ShapeDtypeStruct((NPAD,), jnp.float32),
        mesh=mesh,
        scratch_types=[
            pltpu.VMEM_SHARED((NPAD,), jnp.float32),
            pltpu.VMEM((CH,), jnp.int32),
            pltpu.VMEM((CH,), jnp.float32),
            pltpu.VMEM((SPT,), jnp.float32),
        ],
    )
    def deg_kernel(col_hbm, dis_hbm, deg_sp, idx_v, ones_v, buf_v):
        c = lax.axis_index("c")
        s = lax.axis_index("s")

        def zb(i, carry):
            buf_v[pl.ds(i * _LANES, _LANES)] = jnp.zeros((_LANES,), jnp.float32)
            return carry

        lax.fori_loop(0, SPT // _LANES, zb, 0)
        pltpu.sync_copy(buf_v, deg_sp.at[pl.ds(s * SPT, SPT)])

        def ob(i, carry):
            ones_v[pl.ds(i * _LANES, _LANES)] = jnp.ones((_LANES,), jnp.float32)
            return carry

        lax.fori_loop(0, CH // _LANES, ob, 0)
        plsc.subcore_barrier()

        @pl.when(c == 0)
        def _hist():
            base = s * EPT

            def body(i, carry):
                pltpu.sync_copy(col_hbm.at[pl.ds(base + i * CH, CH)], idx_v)
                pltpu.sync_copy(ones_v, deg_sp.at[idx_v], add=True)
                return carry

            lax.fori_loop(0, ITERS, body, 0)

        plsc.subcore_barrier()

        @pl.when(c == 0)
        def _writeback():
            pltpu.sync_copy(deg_sp.at[pl.ds(s * SPT, SPT)],
                            dis_hbm.at[pl.ds(s * SPT, SPT)])

    return deg_kernel


def _make_mm_kernel(NPAD, D):
    DH = D // 2
    BLK = 1024
    GRID = NPAD // BLK

    def body(x_ref, w_ref, deg_ref, h_ref):
        h = lax.dot_general(
            x_ref[...], w_ref[...], (((1,), (1,)), ((), ())),
            preferred_element_type=jnp.float32,
            precision=lax.Precision.HIGHEST,
        )
        h_ref[...] = h * lax.rsqrt(deg_ref[...] + 1.0)

    return pl.pallas_call(
        body,
        grid=(GRID,),
        in_specs=[
            pl.BlockSpec((BLK, D), lambda i: (i, 0)),
            pl.BlockSpec((D, D), lambda i: (0, 0)),
            pl.BlockSpec((BLK, 1), lambda i: (i, 0)),
        ],
        out_specs=pl.BlockSpec((BLK, D), lambda i: (i, 0)),
        out_shape=jax.ShapeDtypeStruct((NPAD, D), jnp.float32),
    )


def _make_edge_kernel(E, NPAD, D):
    NW = _NC * _NS
    EPW = E // NW
    CH = 80
    ITERS = EPW // CH
    SPT = NPAD // _NS
    ZR = 80
    mesh = plsc.VectorSubcoreMesh(core_axis_name="c", subcore_axis_name="s")

    @functools.partial(
        pl.kernel,
        out_type=(
            jax.ShapeDtypeStruct((NPAD, D), jnp.float32),
            jax.ShapeDtypeStruct((NPAD, D), jnp.float32),
        ),
        mesh=mesh,
        scratch_types=[
            pltpu.VMEM_SHARED((NPAD, D), jnp.float32),
            pltpu.VMEM((ITERS, CH), jnp.int32),
            pltpu.VMEM((ITERS, CH), jnp.int32),
            pltpu.VMEM((CH, D), jnp.float32),
            pltpu.SemaphoreType.DMA,
        ],
    )
    def edge_kernel(row_hbm, col_hbm, h_hbm, p0_hbm, p1_hbm,
                    acc_sp, ridx_v, cidx_v, rows_v, sem):
        c = lax.axis_index("c")
        s = lax.axis_index("s")
        wid = c * _NS + s

        def zb(i, carry):
            def zl(k, carry2):
                rows_v[i, pl.ds(k * _LANES, _LANES)] = jnp.zeros(
                    (_LANES,), jnp.float32)
                return carry2

            lax.fori_loop(0, D // _LANES, zl, 0)
            return carry

        lax.fori_loop(0, CH, zb, 0)

        def zcp(t, carry):
            pltpu.sync_copy(rows_v, acc_sp.at[pl.ds(s * SPT + t * ZR, ZR)])
            return carry

        lax.fori_loop(0, SPT // ZR, zcp, 0)

        pltpu.sync_copy(row_hbm.at[wid], ridx_v)
        pltpu.sync_copy(col_hbm.at[wid], cidx_v)
        plsc.subcore_barrier()

        def body(j, carry):
            pltpu.async_copy(h_hbm.at[ridx_v.at[j]], rows_v, sem).wait()
            pltpu.sync_copy(rows_v, acc_sp.at[cidx_v.at[j]], add=True)
            return carry

        lax.fori_loop(0, ITERS, body, 0)

        plsc.subcore_barrier()

        @pl.when(c == 0)
        def _w0():
            pltpu.sync_copy(acc_sp.at[pl.ds(s * SPT, SPT)],
                            p0_hbm.at[pl.ds(s * SPT, SPT)])

        @pl.when(c == 1)
        def _w1():
            pltpu.sync_copy(acc_sp.at[pl.ds(s * SPT, SPT)],
                            p1_hbm.at[pl.ds(s * SPT, SPT)])

    return edge_kernel


def _make_final_kernel(NPAD, D):
    DH = D // 2
    BLK = 1024
    GRID = NPAD // BLK

    def body(p0_ref, p1_ref, h_ref, deg_ref, out_ref):
        dis = lax.rsqrt(deg_ref[...] + 1.0)
        z = (p0_ref[...] + p1_ref[...] + h_ref[...]) * dis
        out_ref[...] = jnp.where(z > 0, 2.0 * z, z)

    return pl.pallas_call(
        body,
        grid=(GRID,),
        in_specs=[
            pl.BlockSpec((BLK, D), lambda i: (i, 0)),
            pl.BlockSpec((BLK, D), lambda i: (i, 0)),
            pl.BlockSpec((BLK, D), lambda i: (i, 0)),
            pl.BlockSpec((BLK, 1), lambda i: (i, 0)),
        ],
        out_specs=pl.BlockSpec((BLK, D), lambda i: (i, 0)),
        out_shape=jax.ShapeDtypeStruct((NPAD, D), jnp.float32),
    )


def kernel(x, edge_index, edge_attr, W):
    N, D = x.shape
    E = edge_index.shape[1]
    NPAD = ((N + 1023) // 1024) * 1024
    NW = _NC * _NS
    EPW = E // NW

    xp = jnp.zeros((NPAD, D), x.dtype).at[:N].set(x)
    row = edge_index[0]
    col = edge_index[1]
    row3 = row.reshape(NW, EPW // 80, 80)
    col3 = col.reshape(NW, EPW // 80, 80)

    deg = _make_deg_kernel(E, NPAD)(col)
    deg2 = deg.reshape(NPAD, 1)
    h = _make_mm_kernel(NPAD, D)(xp, W, deg2)
    p0, p1 = _make_edge_kernel(E, NPAD, D)(row3, col3, h)
    out = _make_final_kernel(NPAD, D)(p0, p1, h, deg2)
    return out[:N]


# speedup vs baseline: 34.5897x; 1.7766x over previous
import functools

import jax
import jax.numpy as jnp
from jax import lax
from jax.experimental import pallas as pl
from jax.experimental.pallas import tpu as pltpu
from jax.experimental.pallas import tpu_sc as plsc

_NC = 2
_NS = 16
_LANES = 16
_CH = 80


def _make_deg_kernel(E, NPAD):
    NW = _NC * _NS
    EPW = E // NW
    ITERS = EPW // _CH
    SPT = NPAD // _NS
    DEPTH = 4
    mesh = plsc.VectorSubcoreMesh(core_axis_name="c", subcore_axis_name="s")

    @functools.partial(
        pl.kernel,
        out_type=jax.ShapeDtypeStruct((_NC, NPAD), jnp.float32),
        mesh=mesh,
        scratch_types=[
            pltpu.VMEM_SHARED((NPAD,), jnp.float32),
            pltpu.VMEM((ITERS, _CH), jnp.int32),
            pltpu.VMEM((_CH,), jnp.float32),
            pltpu.VMEM((SPT,), jnp.float32),
            pltpu.SemaphoreType.DMA,
        ],
    )
    def deg_kernel(col_hbm, parts_hbm, deg_sp, cidx_v, ones_v, buf_v, sem):
        c = lax.axis_index("c")
        s = lax.axis_index("s")
        wid = c * _NS + s

        def zb(i, carry):
            buf_v[pl.ds(i * _LANES, _LANES)] = jnp.zeros((_LANES,), jnp.float32)
            return carry

        lax.fori_loop(0, SPT // _LANES, zb, 0)
        pltpu.sync_copy(buf_v, deg_sp.at[pl.ds(s * SPT, SPT)])

        def ob(i, carry):
            ones_v[pl.ds(i * _LANES, _LANES)] = jnp.ones((_LANES,), jnp.float32)
            return carry

        lax.fori_loop(0, _CH // _LANES, ob, 0)
        pltpu.sync_copy(col_hbm.at[wid], cidx_v)
        plsc.subcore_barrier()

        def fire(j, carry):
            pltpu.async_copy(ones_v, deg_sp.at[cidx_v.at[j]], sem, add=True)

            @pl.when(j >= DEPTH)
            def _():
                pltpu.make_async_copy(ones_v, deg_sp.at[cidx_v.at[0]],
                                      sem).wait()

            return carry

        lax.fori_loop(0, ITERS, fire, 0)

        def drain(j, carry):
            pltpu.make_async_copy(ones_v, deg_sp.at[cidx_v.at[0]], sem).wait()
            return carry

        lax.fori_loop(0, DEPTH, drain, 0)
        plsc.subcore_barrier()

        pltpu.sync_copy(deg_sp.at[pl.ds(s * SPT, SPT)],
                        parts_hbm.at[c, pl.ds(s * SPT, SPT)])

    return deg_kernel


def _make_mm_kernel(NPAD, D):
    BLK = 1024
    GRID = NPAD // BLK

    def body(x_ref, w_ref, deg_ref, h_ref):
        h = lax.dot_general(
            x_ref[...], w_ref[...], (((1,), (1,)), ((), ())),
            preferred_element_type=jnp.float32,
            precision=lax.Precision.HIGHEST,
        )
        deg = deg_ref[:, 0:1] + deg_ref[:, 1:2]
        h_ref[...] = h * lax.rsqrt(deg + 1.0)

    return pl.pallas_call(
        body,
        grid=(GRID,),
        in_specs=[
            pl.BlockSpec((BLK, D), lambda i: (i, 0)),
            pl.BlockSpec((D, D), lambda i: (0, 0)),
            pl.BlockSpec((BLK, _NC), lambda i: (i, 0)),
        ],
        out_specs=pl.BlockSpec((BLK, D), lambda i: (i, 0)),
        out_shape=jax.ShapeDtypeStruct((NPAD, D), jnp.float32),
    )


def _make_edge_kernel(E, NPAD, D):
    NW = _NC * _NS
    EPW = E // NW
    ITERS = EPW // _CH
    assert ITERS % 2 == 1, "pipeline tail below assumes an odd chunk count"
    HALF = (ITERS - 1) // 2
    SPT = NPAD // _NS
    mesh = plsc.VectorSubcoreMesh(core_axis_name="c", subcore_axis_name="s")

    @functools.partial(
        pl.kernel,
        out_type=(
            jax.ShapeDtypeStruct((NPAD, D), jnp.float32),
            jax.ShapeDtypeStruct((NPAD, D), jnp.float32),
        ),
        mesh=mesh,
        scratch_types=[
            pltpu.VMEM_SHARED((NPAD, D), jnp.float32),
            pltpu.VMEM((ITERS, _CH), jnp.int32),
            pltpu.VMEM((2, _CH), jnp.int32),
            pltpu.VMEM((2, _CH), jnp.int32),
            pltpu.VMEM((_CH, D), jnp.float32),
            pltpu.VMEM((_CH, D), jnp.float32),
            pltpu.SemaphoreType.DMA,
            pltpu.SemaphoreType.DMA,
            pltpu.SemaphoreType.DMA,
            pltpu.SemaphoreType.DMA,
        ],
    )
    def edge_kernel(packed_hbm, h_hbm, p0_hbm, p1_hbm,
                    acc_sp, packed_v, ridx2, cidx2, buf0, buf1, g0, g1, s0, s1):
        c = lax.axis_index("c")
        s = lax.axis_index("s")
        wid = c * _NS + s

        def unpack(j, p):
            def u(k, carry):
                pk = packed_v[j, pl.ds(k * _LANES, _LANES)]
                ridx2[p, pl.ds(k * _LANES, _LANES)] = pk & jnp.int32(0xFFFF)
                cidx2[p, pl.ds(k * _LANES, _LANES)] = lax.shift_right_logical(
                    pk, 16)
                return carry

            lax.fori_loop(0, _CH // _LANES, u, 0)

        def zb(i, carry):
            def zl(k, carry2):
                buf0[i, pl.ds(k * _LANES, _LANES)] = jnp.zeros(
                    (_LANES,), jnp.float32)
                return carry2

            lax.fori_loop(0, D // _LANES, zl, 0)
            return carry

        lax.fori_loop(0, _CH, zb, 0)

        def zcp(t, carry):
            pltpu.sync_copy(buf0, acc_sp.at[pl.ds(s * SPT + t * _CH, _CH)])
            return carry

        lax.fori_loop(0, SPT // _CH, zcp, 0)

        pltpu.sync_copy(packed_hbm.at[wid], packed_v)
        plsc.subcore_barrier()

        def gather(p, buf, sem):
            pltpu.async_copy(h_hbm.at[ridx2.at[p]], buf, sem)

        def scatter(p, buf, sem):
            pltpu.async_copy(buf, acc_sp.at[cidx2.at[p]], sem, add=True)

        def wait_g(buf, sem):
            pltpu.make_async_copy(h_hbm.at[ridx2.at[0]], buf, sem).wait()

        def wait_s(buf, sem):
            pltpu.make_async_copy(buf, acc_sp.at[cidx2.at[0]], sem).wait()

        unpack(0, 0)
        unpack(1, 1)
        gather(0, buf0, g0)
        gather(1, buf1, g1)

        def body(t, carry):
            j0 = 2 * t
            j1 = j0 + 1
            wait_g(buf0, g0)
            scatter(0, buf0, s0)
            wait_g(buf1, g1)
            scatter(1, buf1, s1)
            wait_s(buf0, s0)
            unpack(j0 + 2, 0)
            gather(0, buf0, g0)

            @pl.when(j1 + 2 < ITERS)
            def _():
                wait_s(buf1, s1)
                unpack(j1 + 2, 1)
                gather(1, buf1, g1)

            return carry

        lax.fori_loop(0, HALF, body, 0)
        wait_g(buf0, g0)
        scatter(0, buf0, s0)
        wait_s(buf0, s0)
        wait_s(buf1, s1)

        plsc.subcore_barrier()

        @pl.when(c == 0)
        def _w0():
            pltpu.sync_copy(acc_sp.at[pl.ds(s * SPT, SPT)],
                            p0_hbm.at[pl.ds(s * SPT, SPT)])

        @pl.when(c == 1)
        def _w1():
            pltpu.sync_copy(acc_sp.at[pl.ds(s * SPT, SPT)],
                            p1_hbm.at[pl.ds(s * SPT, SPT)])

    return edge_kernel


def _make_final_kernel(NPAD, D):
    BLK = 1024
    GRID = NPAD // BLK

    def body(p0_ref, p1_ref, h_ref, deg_ref, out_ref):
        deg = deg_ref[:, 0:1] + deg_ref[:, 1:2]
        dis = lax.rsqrt(deg + 1.0)
        z = (p0_ref[...] + p1_ref[...] + h_ref[...]) * dis
        out_ref[...] = jnp.where(z > 0, 2.0 * z, z)

    return pl.pallas_call(
        body,
        grid=(GRID,),
        in_specs=[
            pl.BlockSpec((BLK, D), lambda i: (i, 0)),
            pl.BlockSpec((BLK, D), lambda i: (i, 0)),
            pl.BlockSpec((BLK, D), lambda i: (i, 0)),
            pl.BlockSpec((BLK, _NC), lambda i: (i, 0)),
        ],
        out_specs=pl.BlockSpec((BLK, D), lambda i: (i, 0)),
        out_shape=jax.ShapeDtypeStruct((NPAD, D), jnp.float32),
    )


def kernel(x, edge_index, edge_attr, W):
    N, D = x.shape
    E = edge_index.shape[1]
    NPAD = ((N + 1023) // 1024) * 1024
    NW = _NC * _NS
    EPW = E // NW

    xp = jnp.zeros((NPAD, D), x.dtype).at[:N].set(x)
    row = edge_index[0]
    col = edge_index[1]
    col3 = col.reshape(NW, EPW // _CH, _CH)
    packed = (row | (col << 16)).reshape(NW, EPW // _CH, _CH)

    parts = _make_deg_kernel(E, NPAD)(col3)
    parts_t = parts.T
    h = _make_mm_kernel(NPAD, D)(xp, W, parts_t)
    p0, p1 = _make_edge_kernel(E, NPAD, D)(packed, h)
    out = _make_final_kernel(NPAD, D)(p0, p1, h, parts_t)
    return out[:N]


# speedup vs baseline: 47.3310x; 1.3684x over previous
import functools

import jax
import jax.numpy as jnp
from jax import lax
from jax.experimental import pallas as pl
from jax.experimental.pallas import tpu as pltpu
from jax.experimental.pallas import tpu_sc as plsc

_NC = 2
_NS = 16
_LANES = 16
_CH = 80
_NBUF = 3


def _make_deg_kernel(E, NPAD):
    NW = _NC * _NS
    EPW = E // NW
    ITERS = EPW // _CH
    SPT = NPAD // _NS
    DEPTH = 4
    mesh = plsc.VectorSubcoreMesh(core_axis_name="c", subcore_axis_name="s")

    @functools.partial(
        pl.kernel,
        out_type=jax.ShapeDtypeStruct((_NC, NPAD), jnp.float32),
        mesh=mesh,
        scratch_types=[
            pltpu.VMEM_SHARED((NPAD,), jnp.float32),
            pltpu.VMEM((ITERS, _CH), jnp.int32),
            pltpu.VMEM((_CH,), jnp.float32),
            pltpu.VMEM((SPT,), jnp.float32),
            pltpu.SemaphoreType.DMA,
        ],
    )
    def deg_kernel(col_hbm, parts_hbm, deg_sp, cidx_v, ones_v, buf_v, sem):
        c = lax.axis_index("c")
        s = lax.axis_index("s")
        wid = c * _NS + s

        def zb(i, carry):
            buf_v[pl.ds(i * _LANES, _LANES)] = jnp.zeros((_LANES,), jnp.float32)
            return carry

        lax.fori_loop(0, SPT // _LANES, zb, 0)
        pltpu.sync_copy(buf_v, deg_sp.at[pl.ds(s * SPT, SPT)])

        def ob(i, carry):
            ones_v[pl.ds(i * _LANES, _LANES)] = jnp.ones((_LANES,), jnp.float32)
            return carry

        lax.fori_loop(0, _CH // _LANES, ob, 0)
        pltpu.sync_copy(col_hbm.at[wid], cidx_v)
        plsc.subcore_barrier()

        def fire(j, carry):
            pltpu.async_copy(ones_v, deg_sp.at[cidx_v.at[j]], sem, add=True)

            @pl.when(j >= DEPTH)
            def _():
                pltpu.make_async_copy(ones_v, deg_sp.at[cidx_v.at[0]],
                                      sem).wait()

            return carry

        lax.fori_loop(0, ITERS, fire, 0)

        def drain(j, carry):
            pltpu.make_async_copy(ones_v, deg_sp.at[cidx_v.at[0]], sem).wait()
            return carry

        lax.fori_loop(0, DEPTH, drain, 0)
        plsc.subcore_barrier()

        pltpu.sync_copy(deg_sp.at[pl.ds(s * SPT, SPT)],
                        parts_hbm.at[c, pl.ds(s * SPT, SPT)])

    return deg_kernel


def _make_mm_kernel(N, NPAD, D):
    BLK = 1000
    GRID = N // BLK

    def body(x_ref, w_ref, deg_ref, h_ref):
        h = lax.dot_general(
            x_ref[...], w_ref[...], (((1,), (1,)), ((), ())),
            preferred_element_type=jnp.float32,
            precision=lax.Precision.HIGHEST,
        )
        deg = deg_ref[:, 0:1] + deg_ref[:, 1:2]
        h_ref[...] = h * lax.rsqrt(deg + 1.0)

    return pl.pallas_call(
        body,
        grid=(GRID,),
        in_specs=[
            pl.BlockSpec((BLK, D), lambda i: (i, 0)),
            pl.BlockSpec((D, D), lambda i: (0, 0)),
            pl.BlockSpec((BLK, _NC), lambda i: (i, 0)),
        ],
        out_specs=pl.BlockSpec((BLK, D), lambda i: (i, 0)),
        out_shape=jax.ShapeDtypeStruct((N, D), jnp.float32),
    )


def _make_edge_kernel(E, NPAD, D):
    NW = _NC * _NS
    EPW = E // NW
    ITERS = EPW // _CH
    SPT = NPAD // _NS
    ZFULL = SPT // _CH
    ZREM = SPT - ZFULL * _CH
    TQ = (ITERS + _NBUF - 1) // _NBUF
    mesh = plsc.VectorSubcoreMesh(core_axis_name="c", subcore_axis_name="s")

    @functools.partial(
        pl.kernel,
        out_type=(
            jax.ShapeDtypeStruct((NPAD, D), jnp.float32),
            jax.ShapeDtypeStruct((NPAD, D), jnp.float32),
        ),
        mesh=mesh,
        scratch_types=[
            pltpu.VMEM_SHARED((NPAD, D), jnp.float32),
            pltpu.VMEM((ITERS, _CH), jnp.int32),
            pltpu.VMEM((_NBUF, _CH), jnp.int32),
            pltpu.VMEM((_NBUF, _CH), jnp.int32),
        ] + [pltpu.VMEM((_CH, D), jnp.float32)] * _NBUF
          + [pltpu.SemaphoreType.DMA] * (2 * _NBUF),
    )
    def edge_kernel(packed_hbm, h_hbm, p0_hbm, p1_hbm,
                    acc_sp, packed_v, ridx2, cidx2, *bufsem):
        bufs = bufsem[:_NBUF]
        gsem = bufsem[_NBUF:2 * _NBUF]
        ssem = bufsem[2 * _NBUF:]
        c = lax.axis_index("c")
        s = lax.axis_index("s")
        wid = c * _NS + s

        def unpack(j, p):
            def u(k, carry):
                pk = packed_v[j, pl.ds(k * _LANES, _LANES)]
                ridx2[p, pl.ds(k * _LANES, _LANES)] = pk & jnp.int32(0xFFFF)
                cidx2[p, pl.ds(k * _LANES, _LANES)] = lax.shift_right_logical(
                    pk, 16)
                return carry

            lax.fori_loop(0, _CH // _LANES, u, 0)

        def zb(i, carry):
            def zl(k, carry2):
                bufs[0][i, pl.ds(k * _LANES, _LANES)] = jnp.zeros(
                    (_LANES,), jnp.float32)
                return carry2

            lax.fori_loop(0, D // _LANES, zl, 0)
            return carry

        lax.fori_loop(0, _CH, zb, 0)

        def zcp(t, carry):
            pltpu.sync_copy(bufs[0], acc_sp.at[pl.ds(s * SPT + t * _CH, _CH)])
            return carry

        lax.fori_loop(0, ZFULL, zcp, 0)
        if ZREM:
            pltpu.sync_copy(
                bufs[0].at[pl.ds(0, ZREM)],
                acc_sp.at[pl.ds(s * SPT + ZFULL * _CH, ZREM)])

        pltpu.sync_copy(packed_hbm.at[wid], packed_v)
        plsc.subcore_barrier()

        def gather(b):
            pltpu.async_copy(h_hbm.at[ridx2.at[b]], bufs[b], gsem[b])

        def scatter(b):
            pltpu.async_copy(bufs[b], acc_sp.at[cidx2.at[b]], ssem[b],
                             add=True)

        def wait_g(b):
            pltpu.make_async_copy(h_hbm.at[ridx2.at[b]], bufs[b],
                                  gsem[b]).wait()

        def wait_s(b):
            pltpu.make_async_copy(bufs[b], acc_sp.at[cidx2.at[b]],
                                  ssem[b]).wait()

        for b in range(_NBUF):
            unpack(b, b)
            gather(b)

        def body(t, carry):
            for b in range(_NBUF):
                j = _NBUF * t + b

                @pl.when(j < ITERS)
                def _(b=b, j=j):
                    wait_g(b)
                    scatter(b)

                    @pl.when(j + _NBUF < ITERS)
                    def _():
                        wait_s(b)
                        unpack(j + _NBUF, b)
                        gather(b)

            return carry

        lax.fori_loop(0, TQ, body, 0)
        for b in range(_NBUF):
            wait_s(b)

        plsc.subcore_barrier()

        @pl.when(c == 0)
        def _w0():
            pltpu.sync_copy(acc_sp.at[pl.ds(s * SPT, SPT)],
                            p0_hbm.at[pl.ds(s * SPT, SPT)])

        @pl.when(c == 1)
        def _w1():
            pltpu.sync_copy(acc_sp.at[pl.ds(s * SPT, SPT)],
                            p1_hbm.at[pl.ds(s * SPT, SPT)])

    return edge_kernel


def _make_final_kernel(N, NPAD, D):
    BLK = 1000
    GRID = N // BLK

    def body(p0_ref, p1_ref, h_ref, deg_ref, out_ref):
        deg = deg_ref[:, 0:1] + deg_ref[:, 1:2]
        dis = lax.rsqrt(deg + 1.0)
        z = (p0_ref[...] + p1_ref[...] + h_ref[...]) * dis
        out_ref[...] = jnp.where(z > 0, 2.0 * z, z)

    return pl.pallas_call(
        body,
        grid=(GRID,),
        in_specs=[
            pl.BlockSpec((BLK, D), lambda i: (i, 0)),
            pl.BlockSpec((BLK, D), lambda i: (i, 0)),
            pl.BlockSpec((BLK, D), lambda i: (i, 0)),
            pl.BlockSpec((BLK, _NC), lambda i: (i, 0)),
        ],
        out_specs=pl.BlockSpec((BLK, D), lambda i: (i, 0)),
        out_shape=jax.ShapeDtypeStruct((N, D), jnp.float32),
    )


def kernel(x, edge_index, edge_attr, W):
    N, D = x.shape
    E = edge_index.shape[1]
    NPAD = ((N + 1023) // 1024) * 1024
    NW = _NC * _NS
    EPW = E // NW

    row = edge_index[0]
    col = edge_index[1]
    col3 = col.reshape(NW, EPW // _CH, _CH)
    packed = (row | (col << 16)).reshape(NW, EPW // _CH, _CH)

    parts = _make_deg_kernel(E, NPAD)(col3)
    parts_t = parts.T
    h = _make_mm_kernel(N, NPAD, D)(x, W, parts_t)
    p0, p1 = _make_edge_kernel(E, NPAD, D)(packed, h)
    return _make_final_kernel(N, NPAD, D)(p0, p1, h, parts_t)


# speedup vs baseline: 48.7048x; 1.0290x over previous
import functools

import jax
import jax.numpy as jnp
from jax import lax
from jax.experimental import pallas as pl
from jax.experimental.pallas import tpu as pltpu
from jax.experimental.pallas import tpu_sc as plsc

_NC = 2
_NS = 16
_LANES = 16
_CH = 80
_NBUF = 4


def _make_deg_kernel(E, NPAD):
    NW = _NC * _NS
    EPW = E // NW
    ITERS = EPW // _CH
    SPT = NPAD // _NS
    DEPTH = 4
    mesh = plsc.VectorSubcoreMesh(core_axis_name="c", subcore_axis_name="s")

    @functools.partial(
        pl.kernel,
        out_type=jax.ShapeDtypeStruct((_NC, NPAD), jnp.float32),
        mesh=mesh,
        scratch_types=[
            pltpu.VMEM_SHARED((NPAD,), jnp.float32),
            pltpu.VMEM((ITERS, _CH), jnp.int32),
            pltpu.VMEM((_CH,), jnp.float32),
            pltpu.VMEM((SPT,), jnp.float32),
            pltpu.SemaphoreType.DMA,
        ],
    )
    def deg_kernel(col_hbm, parts_hbm, deg_sp, cidx_v, ones_v, buf_v, sem):
        c = lax.axis_index("c")
        s = lax.axis_index("s")
        wid = c * _NS + s

        def zb(i, carry):
            buf_v[pl.ds(i * _LANES, _LANES)] = jnp.zeros((_LANES,), jnp.float32)
            return carry

        lax.fori_loop(0, SPT // _LANES, zb, 0)
        pltpu.sync_copy(buf_v, deg_sp.at[pl.ds(s * SPT, SPT)])

        def ob(i, carry):
            ones_v[pl.ds(i * _LANES, _LANES)] = jnp.ones((_LANES,), jnp.float32)
            return carry

        lax.fori_loop(0, _CH // _LANES, ob, 0)
        pltpu.sync_copy(col_hbm.at[wid], cidx_v)
        plsc.subcore_barrier()

        def fire(j, carry):
            pltpu.async_copy(ones_v, deg_sp.at[cidx_v.at[j]], sem, add=True)

            @pl.when(j >= DEPTH)
            def _():
                pltpu.make_async_copy(ones_v, deg_sp.at[cidx_v.at[0]],
                                      sem).wait()

            return carry

        lax.fori_loop(0, ITERS, fire, 0)

        def drain(j, carry):
            pltpu.make_async_copy(ones_v, deg_sp.at[cidx_v.at[0]], sem).wait()
            return carry

        lax.fori_loop(0, DEPTH, drain, 0)
        plsc.subcore_barrier()

        pltpu.sync_copy(deg_sp.at[pl.ds(s * SPT, SPT)],
                        parts_hbm.at[c, pl.ds(s * SPT, SPT)])

    return deg_kernel


def _make_mm_kernel(N, NPAD, D):
    BLK = 1000
    GRID = N // BLK

    def body(x_ref, w_ref, deg_ref, h_ref):
        h = lax.dot_general(
            x_ref[...], w_ref[...], (((1,), (1,)), ((), ())),
            preferred_element_type=jnp.float32,
            precision=lax.Precision.HIGHEST,
        )
        deg = deg_ref[:, 0:1] + deg_ref[:, 1:2]
        h_ref[...] = h * lax.rsqrt(deg + 1.0)

    return pl.pallas_call(
        body,
        grid=(GRID,),
        in_specs=[
            pl.BlockSpec((BLK, D), lambda i: (i, 0)),
            pl.BlockSpec((D, D), lambda i: (0, 0)),
            pl.BlockSpec((BLK, _NC), lambda i: (i, 0)),
        ],
        out_specs=pl.BlockSpec((BLK, D), lambda i: (i, 0)),
        out_shape=jax.ShapeDtypeStruct((N, D), jnp.float32),
    )


def _make_edge_kernel(E, NPAD, D):
    NW = _NC * _NS
    EPW = E // NW
    ITERS = EPW // _CH
    SPT = NPAD // _NS
    ZFULL = SPT // _CH
    ZREM = SPT - ZFULL * _CH
    mesh = plsc.VectorSubcoreMesh(core_axis_name="c", subcore_axis_name="s")

    @functools.partial(
        pl.kernel,
        out_type=(
            jax.ShapeDtypeStruct((NPAD, D), jnp.float32),
            jax.ShapeDtypeStruct((NPAD, D), jnp.float32),
        ),
        mesh=mesh,
        scratch_types=[
            pltpu.VMEM_SHARED((NPAD, D), jnp.float32),
            pltpu.VMEM((2 * _NBUF, _CH), jnp.int32),
            pltpu.VMEM((2 * _NBUF, _CH), jnp.int32),
        ] + [pltpu.VMEM((_CH, D), jnp.float32)] * _NBUF
          + [pltpu.SemaphoreType.DMA] * (4 * _NBUF),
    )
    def edge_kernel(row_hbm, col_hbm, h_hbm, p0_hbm, p1_hbm,
                    acc_sp, ridx2, cidx2, *bufsem):
        bufs = bufsem[:_NBUF]
        gsem = bufsem[_NBUF:2 * _NBUF]
        ssem = bufsem[2 * _NBUF:3 * _NBUF]
        isem = bufsem[3 * _NBUF:]
        c = lax.axis_index("c")
        s = lax.axis_index("s")
        wid = c * _NS + s

        def load_idx(j, q):
            pltpu.async_copy(row_hbm.at[wid, j], ridx2.at[q], isem[q])
            pltpu.async_copy(col_hbm.at[wid, j], cidx2.at[q], isem[q])

        def wait_idx(q):
            pltpu.make_async_copy(row_hbm.at[wid, 0], ridx2.at[q],
                                  isem[q]).wait()
            pltpu.make_async_copy(col_hbm.at[wid, 0], cidx2.at[q],
                                  isem[q]).wait()

        def zb(i, carry):
            def zl(k, carry2):
                bufs[0][i, pl.ds(k * _LANES, _LANES)] = jnp.zeros(
                    (_LANES,), jnp.float32)
                return carry2

            lax.fori_loop(0, D // _LANES, zl, 0)
            return carry

        lax.fori_loop(0, _CH, zb, 0)

        def zcp(t, carry):
            pltpu.sync_copy(bufs[0], acc_sp.at[pl.ds(s * SPT + t * _CH, _CH)])
            return carry

        lax.fori_loop(0, ZFULL, zcp, 0)
        if ZREM:
            pltpu.sync_copy(
                bufs[0].at[pl.ds(0, ZREM)],
                acc_sp.at[pl.ds(s * SPT + ZFULL * _CH, ZREM)])

        plsc.subcore_barrier()

        def gather(b, q):
            pltpu.async_copy(h_hbm.at[ridx2.at[q]], bufs[b], gsem[b])

        def scatter(b, q):
            pltpu.async_copy(bufs[b], acc_sp.at[cidx2.at[q]], ssem[b],
                             add=True)

        def wait_g(b):
            pltpu.make_async_copy(h_hbm.at[ridx2.at[0]], bufs[b],
                                  gsem[b]).wait()

        def wait_s(b):
            pltpu.make_async_copy(bufs[b], acc_sp.at[cidx2.at[0]],
                                  ssem[b]).wait()

        for j in range(2 * _NBUF):
            load_idx(j, 2 * (j % _NBUF) + (j // _NBUF))
        for b in range(_NBUF):
            wait_idx(2 * b)
            gather(b, 2 * b)

        def body(t, carry):
            for u in range(2 * _NBUF):
                b = u % _NBUF
                par = u // _NBUF
                q = 2 * b + par
                qn = 2 * b + (1 - par)
                j = 2 * _NBUF * t + u

                @pl.when(j < ITERS)
                def _(b=b, j=j, q=q, qn=qn):
                    wait_g(b)
                    scatter(b, q)

                    @pl.when(j + _NBUF < ITERS)
                    def _():
                        wait_s(b)

                        @pl.when(j + 2 * _NBUF < ITERS)
                        def _():
                            load_idx(j + 2 * _NBUF, q)

                        wait_idx(qn)
                        gather(b, qn)

            return carry

        lax.fori_loop(0, (ITERS + 2 * _NBUF - 1) // (2 * _NBUF), body, 0)
        for b in range(_NBUF):
            wait_s(b)

        plsc.subcore_barrier()

        @pl.when(c == 0)
        def _w0():
            pltpu.sync_copy(acc_sp.at[pl.ds(s * SPT, SPT)],
                            p0_hbm.at[pl.ds(s * SPT, SPT)])

        @pl.when(c == 1)
        def _w1():
            pltpu.sync_copy(acc_sp.at[pl.ds(s * SPT, SPT)],
                            p1_hbm.at[pl.ds(s * SPT, SPT)])

    return edge_kernel


def _make_final_kernel(N, NPAD, D):
    BLK = 1000
    GRID = N // BLK

    def body(p0_ref, p1_ref, h_ref, deg_ref, out_ref):
        deg = deg_ref[:, 0:1] + deg_ref[:, 1:2]
        dis = lax.rsqrt(deg + 1.0)
        z = (p0_ref[...] + p1_ref[...] + h_ref[...]) * dis
        out_ref[...] = jnp.where(z > 0, 2.0 * z, z)

    return pl.pallas_call(
        body,
        grid=(GRID,),
        in_specs=[
            pl.BlockSpec((BLK, D), lambda i: (i, 0)),
            pl.BlockSpec((BLK, D), lambda i: (i, 0)),
            pl.BlockSpec((BLK, D), lambda i: (i, 0)),
            pl.BlockSpec((BLK, _NC), lambda i: (i, 0)),
        ],
        out_specs=pl.BlockSpec((BLK, D), lambda i: (i, 0)),
        out_shape=jax.ShapeDtypeStruct((N, D), jnp.float32),
    )


def kernel(x, edge_index, edge_attr, W):
    N, D = x.shape
    E = edge_index.shape[1]
    NPAD = ((N + 1023) // 1024) * 1024
    NW = _NC * _NS
    EPW = E // NW

    row3 = edge_index[0].reshape(NW, EPW // _CH, _CH)
    col3 = edge_index[1].reshape(NW, EPW // _CH, _CH)

    parts = _make_deg_kernel(E, NPAD)(col3)
    parts_t = parts.T
    h = _make_mm_kernel(N, NPAD, D)(x, W, parts_t)
    p0, p1 = _make_edge_kernel(E, NPAD, D)(row3, col3, h)
    return _make_final_kernel(N, NPAD, D)(p0, p1, h, parts_t)


# speedup vs baseline: 52.6715x; 1.0814x over previous
import functools

import jax
import jax.numpy as jnp
from jax import lax
from jax.experimental import pallas as pl
from jax.experimental.pallas import tpu as pltpu
from jax.experimental.pallas import tpu_sc as plsc

_NC = 2
_NS = 16
_LANES = 16
_CH = 80
_NBUF = 4


def _make_deg_kernel(E, NPAD):
    NW = _NC * _NS
    EPW = E // NW
    ITERS = EPW // _CH
    SPT = NPAD // _NS
    DEPTH = 4
    mesh = plsc.VectorSubcoreMesh(core_axis_name="c", subcore_axis_name="s")

    @functools.partial(
        pl.kernel,
        out_type=jax.ShapeDtypeStruct((_NC, NPAD), jnp.float32),
        mesh=mesh,
        scratch_types=[
            pltpu.VMEM_SHARED((NPAD,), jnp.float32),
            pltpu.VMEM((EPW,), jnp.int32),
            pltpu.VMEM((ITERS, _CH), jnp.int32),
            pltpu.VMEM((_CH,), jnp.float32),
            pltpu.VMEM((SPT,), jnp.float32),
            pltpu.SemaphoreType.DMA,
        ],
    )
    def deg_kernel(edge_hbm, parts_hbm, deg_sp, flat_v, cidx_v, ones_v,
                   buf_v, sem):
        c = lax.axis_index("c")
        s = lax.axis_index("s")
        wid = c * _NS + s

        def zb(i, carry):
            buf_v[pl.ds(i * _LANES, _LANES)] = jnp.zeros((_LANES,), jnp.float32)
            return carry

        lax.fori_loop(0, SPT // _LANES, zb, 0)
        pltpu.sync_copy(buf_v, deg_sp.at[pl.ds(s * SPT, SPT)])

        def ob(i, carry):
            ones_v[pl.ds(i * _LANES, _LANES)] = jnp.ones((_LANES,), jnp.float32)
            return carry

        lax.fori_loop(0, _CH // _LANES, ob, 0)
        pltpu.sync_copy(edge_hbm.at[pl.ds(E + wid * EPW, EPW)], flat_v)
        PER_ROW = _CH // _LANES

        def rp(i, carry):
            v = flat_v[pl.ds(i * _LANES, _LANES)]
            cidx_v[i // PER_ROW,
                   pl.ds(lax.rem(i, PER_ROW) * _LANES, _LANES)] = v
            return carry

        lax.fori_loop(0, EPW // _LANES, rp, 0)
        plsc.subcore_barrier()

        def fire(j, carry):
            pltpu.async_copy(ones_v, deg_sp.at[cidx_v.at[j]], sem, add=True)

            @pl.when(j >= DEPTH)
            def _():
                pltpu.make_async_copy(ones_v, deg_sp.at[cidx_v.at[0]],
                                      sem).wait()

            return carry

        lax.fori_loop(0, ITERS, fire, 0)

        def drain(j, carry):
            pltpu.make_async_copy(ones_v, deg_sp.at[cidx_v.at[0]], sem).wait()
            return carry

        lax.fori_loop(0, DEPTH, drain, 0)
        plsc.subcore_barrier()

        pltpu.sync_copy(deg_sp.at[pl.ds(s * SPT, SPT)],
                        parts_hbm.at[c, pl.ds(s * SPT, SPT)])

    return deg_kernel


def _make_mm_kernel(N, NPAD, D):
    BLK = 2000
    GRID = N // BLK

    def body(x_ref, w_ref, deg_ref, h_ref):
        h = lax.dot_general(
            x_ref[...], w_ref[...], (((1,), (1,)), ((), ())),
            preferred_element_type=jnp.float32,
        )
        deg = deg_ref[:, 0:1] + deg_ref[:, 1:2]
        h_ref[...] = h * lax.rsqrt(deg + 1.0)

    return pl.pallas_call(
        body,
        grid=(GRID,),
        in_specs=[
            pl.BlockSpec((BLK, D), lambda i: (i, 0)),
            pl.BlockSpec((D, D), lambda i: (0, 0)),
            pl.BlockSpec((BLK, _NC), lambda i: (i, 0)),
        ],
        out_specs=pl.BlockSpec((BLK, D), lambda i: (i, 0)),
        out_shape=jax.ShapeDtypeStruct((N, D), jnp.float32),
    )


def _make_edge_kernel(E, N, NPAD, D):
    NW = _NC * _NS
    EPW = E // NW
    ITERS = EPW // _CH
    SPT = NPAD // _NS
    ZFULL = SPT // _CH
    ZREM = SPT - ZFULL * _CH
    mesh = plsc.VectorSubcoreMesh(core_axis_name="c", subcore_axis_name="s")

    @functools.partial(
        pl.kernel,
        out_type=(
            jax.ShapeDtypeStruct((NPAD, D), jnp.float32),
            jax.ShapeDtypeStruct((NPAD, D), jnp.float32),
        ),
        mesh=mesh,
        scratch_types=[
            pltpu.VMEM_SHARED((NPAD, D), jnp.float32),
            pltpu.VMEM((2 * _NBUF, _CH), jnp.int32),
            pltpu.VMEM((2 * _NBUF, _CH), jnp.int32),
        ] + [pltpu.VMEM((_CH, D), jnp.float32)] * _NBUF
          + [pltpu.SemaphoreType.DMA] * (4 * _NBUF),
    )
    def edge_kernel(edge_hbm, h_hbm, p0_hbm, p1_hbm,
                    acc_sp, ridx2, cidx2, *bufsem):
        bufs = bufsem[:_NBUF]
        gsem = bufsem[_NBUF:2 * _NBUF]
        ssem = bufsem[2 * _NBUF:3 * _NBUF]
        isem = bufsem[3 * _NBUF:]
        c = lax.axis_index("c")
        s = lax.axis_index("s")
        wid = c * _NS + s
        ebase = wid * EPW

        def load_idx(j, q):
            pltpu.async_copy(edge_hbm.at[pl.ds(ebase + j * _CH, _CH)],
                             ridx2.at[q], isem[q])
            pltpu.async_copy(edge_hbm.at[pl.ds(E + ebase + j * _CH, _CH)],
                             cidx2.at[q], isem[q])

        def wait_idx(q):
            pltpu.make_async_copy(edge_hbm.at[pl.ds(0, _CH)],
                                  ridx2.at[q], isem[q]).wait()
            pltpu.make_async_copy(edge_hbm.at[pl.ds(0, _CH)],
                                  cidx2.at[q], isem[q]).wait()

        nchunks = jnp.maximum(
            0, jnp.minimum(SPT, jnp.int32(N) - s * SPT)) // _CH

        @pl.when(c == 0)
        def _ih():
            def hcp(t, carry):
                r0 = s * SPT + t * _CH
                pltpu.sync_copy(h_hbm.at[pl.ds(r0, _CH)],
                                acc_sp.at[pl.ds(r0, _CH)])
                return carry

            lax.fori_loop(0, nchunks, hcp, 0)

        @pl.when(c == 1)
        def _iz():
            def zb(i, carry):
                def zl(k, carry2):
                    bufs[0][i, pl.ds(k * _LANES, _LANES)] = jnp.zeros(
                        (_LANES,), jnp.float32)
                    return carry2

                lax.fori_loop(0, D // _LANES, zl, 0)
                return carry

            lax.fori_loop(0, _CH, zb, 0)

            def zcp(t, carry):
                pltpu.sync_copy(bufs[0],
                                acc_sp.at[pl.ds(s * SPT + t * _CH, _CH)])
                return carry

            lax.fori_loop(0, nchunks, zcp, 0)

        plsc.subcore_barrier()

        def gather(b, q):
            pltpu.async_copy(h_hbm.at[ridx2.at[q]], bufs[b], gsem[b])

        def scatter(b, q):
            pltpu.async_copy(bufs[b], acc_sp.at[cidx2.at[q]], ssem[b],
                             add=True)

        def wait_g(b):
            pltpu.make_async_copy(h_hbm.at[ridx2.at[0]], bufs[b],
                                  gsem[b]).wait()

        def wait_s(b):
            pltpu.make_async_copy(bufs[b], acc_sp.at[cidx2.at[0]],
                                  ssem[b]).wait()

        for j in range(2 * _NBUF):
            load_idx(j, 2 * (j % _NBUF) + (j // _NBUF))
        for b in range(_NBUF):
            wait_idx(2 * b)
            gather(b, 2 * b)

        def body(t, carry):
            for u in range(2 * _NBUF):
                b = u % _NBUF
                par = u // _NBUF
                q = 2 * b + par
                qn = 2 * b + (1 - par)
                j = 2 * _NBUF * t + u

                @pl.when(j < ITERS)
                def _(b=b, j=j, q=q, qn=qn):
                    wait_g(b)
                    scatter(b, q)

                    @pl.when(j + _NBUF < ITERS)
                    def _():
                        wait_s(b)

                        @pl.when(j + 2 * _NBUF < ITERS)
                        def _():
                            load_idx(j + 2 * _NBUF, q)

                        wait_idx(qn)
                        gather(b, qn)

            return carry

        lax.fori_loop(0, (ITERS + 2 * _NBUF - 1) // (2 * _NBUF), body, 0)
        for b in range(_NBUF):
            wait_s(b)

        plsc.subcore_barrier()

        @pl.when(c == 0)
        def _w0():
            pltpu.sync_copy(acc_sp.at[pl.ds(s * SPT, SPT)],
                            p0_hbm.at[pl.ds(s * SPT, SPT)])

        @pl.when(c == 1)
        def _w1():
            pltpu.sync_copy(acc_sp.at[pl.ds(s * SPT, SPT)],
                            p1_hbm.at[pl.ds(s * SPT, SPT)])

    return edge_kernel


def _make_final_kernel(N, NPAD, D):
    BLK = 2000
    GRID = N // BLK

    def body(p0_ref, p1_ref, deg_ref, out_ref):
        deg = deg_ref[:, 0:1] + deg_ref[:, 1:2]
        dis = lax.rsqrt(deg + 1.0)
        z = (p0_ref[...] + p1_ref[...]) * dis
        out_ref[...] = jnp.where(z > 0, 2.0 * z, z)

    return pl.pallas_call(
        body,
        grid=(GRID,),
        in_specs=[
            pl.BlockSpec((BLK, D), lambda i: (i, 0)),
            pl.BlockSpec((BLK, D), lambda i: (i, 0)),
            pl.BlockSpec((BLK, _NC), lambda i: (i, 0)),
        ],
        out_specs=pl.BlockSpec((BLK, D), lambda i: (i, 0)),
        out_shape=jax.ShapeDtypeStruct((N, D), jnp.float32),
    )


def kernel(x, edge_index, edge_attr, W):
    N, D = x.shape
    E = edge_index.shape[1]
    NPAD = ((N + 1023) // 1024) * 1024
    NW = _NC * _NS
    EPW = E // NW

    eflat = edge_index.reshape(2 * E)
    parts = _make_deg_kernel(E, NPAD)(eflat)
    parts_t = parts.T
    h = _make_mm_kernel(N, NPAD, D)(x, W, parts_t)
    p0, p1 = _make_edge_kernel(E, N, NPAD, D)(eflat, h)
    return _make_final_kernel(N, NPAD, D)(p0, p1, parts_t)
